# Initial kernel scaffold; baseline (speedup 1.0000x reference)
import jax
import jax.numpy as jnp
from jax.experimental import pallas as pl


def kernel(x, mean, std):
    raise NotImplementedError("write your pallas kernel here")


# speedup vs baseline: 12451.8558x; 12451.8558x over previous
import functools

import jax
import jax.numpy as jnp
from jax.experimental import pallas as pl
from jax.experimental.pallas import tpu as pltpu
from jax.scipy.stats import norm as _jnorm

_BITS = 4
_SIGMA_CLIP = 2.1
_NLEV = 2 ** _BITS


def _tables(mean, std):
    z0 = -mean / (std + 1e-10)
    cdf_0 = _jnorm.cdf(z0)
    cdf_max = _jnorm.cdf(jnp.asarray(_SIGMA_CLIP, dtype=jnp.float32))
    pos_mass = cdf_max - cdf_0
    t = jnp.linspace(1.0 / (_NLEV - 1), 1.0, _NLEV - 1)
    target = cdf_0 + pos_mass * t
    z_vals = _jnorm.ppf(target)
    q = jnp.concatenate(
        [jnp.zeros((1,), jnp.float32), (z_vals * std + mean).astype(jnp.float32)]
    )
    edges = 0.5 * (q[1:] + q[:-1])
    return q, edges


def _quant_block(edges_ref, q_ref, x_ref, o_ref):
    x = x_ref[...]
    acc = jnp.full(x.shape, q_ref[0], dtype=jnp.float32)
    for j in range(_NLEV - 1):
        acc = jnp.where(x > edges_ref[j], q_ref[j + 1], acc)
    o_ref[...] = acc


def kernel(x, mean, std):
    q, edges = _tables(mean, std)
    orig_shape = x.shape
    n = x.size
    rows = n // 4096
    x2 = x.reshape(rows, 4096)
    block_rows = 512
    grid = (rows // block_rows,)
    out = pl.pallas_call(
        _quant_block,
        grid=grid,
        in_specs=[
            pl.BlockSpec(memory_space=pltpu.SMEM),
            pl.BlockSpec(memory_space=pltpu.SMEM),
            pl.BlockSpec((block_rows, 4096), lambda i: (i, 0)),
        ],
        out_specs=pl.BlockSpec((block_rows, 4096), lambda i: (i, 0)),
        out_shape=jax.ShapeDtypeStruct((rows, 4096), jnp.float32),
        compiler_params=pltpu.CompilerParams(
            dimension_semantics=("arbitrary",),
        ),
    )(edges, q, x2)
    return out.reshape(orig_shape)
